# baseline (speedup 1.0000x reference)
import jax
import jax.numpy as jnp
from jax.experimental import pallas as pl
from jax.experimental.pallas import tpu as pltpu

K = 17
N = 50000
IC = 32
OC = 32
L = 2048
NO = N + K - 1
G = (NO + L - 1) // L
HALO = 128


def _body(w_ref, b_ref, *refs):
    cur = refs[:K]
    prev = refs[K:2 * K]
    out_ref = refs[2 * K]

    i = pl.program_id(0)
    num = pl.num_programs(0)

    def compute(masked):
        acc = None
        for k in range(K):
            z = jnp.concatenate(
                [prev[k][:, HALO - (K - 1):], cur[k][...]], axis=1)
            sh = jax.lax.slice(z, (0, K - 1 - k), (IC, K - 1 - k + L))
            if masked:
                mcol = jax.lax.broadcasted_iota(jnp.int32, (IC, L), 1) + i * L
                valid = jnp.logical_and(mcol >= k, mcol <= (N - 1) + k)
                sh = jnp.where(valid, sh, 0.0)
            p = jax.lax.dot_general(
                w_ref[k], sh, (((1,), (0,)), ((), ())),
                preferred_element_type=jnp.float32)
            acc = p if acc is None else acc + p
        if masked:
            mcol = jax.lax.broadcasted_iota(jnp.int32, (IC, L), 1) + i * L
            for k in range(K):
                vk = jnp.logical_and(mcol >= k, mcol <= (N - 1) + k)
                acc = acc + jnp.where(vk, b_ref[k][:, None], 0.0)
        else:
            acc = acc + jnp.sum(b_ref[...], axis=0)[:, None]
        out_ref[...] = acc

    @pl.when(jnp.logical_and(i > 0, i < num - 1))
    def _fast():
        compute(False)

    @pl.when(jnp.logical_or(i == 0, i == num - 1))
    def _edge():
        compute(True)


def kernel(weights, bias, input_0, input_1, input_2, input_3, input_4,
           input_5, input_6, input_7, input_8, input_9, input_10, input_11,
           input_12, input_13, input_14, input_15, input_16):
    ins = (input_0, input_1, input_2, input_3, input_4, input_5, input_6,
           input_7, input_8, input_9, input_10, input_11, input_12, input_13,
           input_14, input_15, input_16)
    xts = tuple(jnp.transpose(x) for x in ins)

    cur_spec = pl.BlockSpec((IC, L), lambda i: (0, i))
    prev_spec = pl.BlockSpec(
        (IC, HALO), lambda i: (0, jnp.maximum(i * (L // HALO) - 1, 0)))
    outt = pl.pallas_call(
        _body,
        grid=(G,),
        in_specs=[
            pl.BlockSpec((K, OC, IC), lambda i: (0, 0, 0)),
            pl.BlockSpec((K, OC), lambda i: (0, 0)),
        ] + [cur_spec] * K + [prev_spec] * K,
        out_specs=pl.BlockSpec((OC, L), lambda i: (0, i)),
        out_shape=jax.ShapeDtypeStruct((OC, NO), jnp.float32),
        compiler_params=pltpu.CompilerParams(
            dimension_semantics=("arbitrary",)),
    )(weights, bias, *xts, *xts)
    return jnp.transpose(outt)


# speedup vs baseline: 1.2203x; 1.2203x over previous
import jax
import jax.numpy as jnp
from jax.experimental import pallas as pl
from jax.experimental.pallas import tpu as pltpu

K = 17
N = 50000
IC = 32
OC = 32
L = 8192
NO = N + K - 1
G = (NO + L - 1) // L
HALO = 128


def _body(w_ref, b_ref, *refs):
    cur = refs[:K]
    prev = refs[K:2 * K]
    out_ref = refs[2 * K]

    i = pl.program_id(0)
    num = pl.num_programs(0)

    def compute(masked):
        acc = None
        for k in range(K):
            z = jnp.concatenate(
                [prev[k][:, HALO - (K - 1):], cur[k][...]], axis=1)
            sh = jax.lax.slice(z, (0, K - 1 - k), (IC, K - 1 - k + L))
            if masked:
                mcol = jax.lax.broadcasted_iota(jnp.int32, (IC, L), 1) + i * L
                valid = jnp.logical_and(mcol >= k, mcol <= (N - 1) + k)
                sh = jnp.where(valid, sh, 0.0)
            p = jax.lax.dot_general(
                w_ref[k], sh, (((1,), (0,)), ((), ())),
                preferred_element_type=jnp.float32)
            acc = p if acc is None else acc + p
        if masked:
            mcol = jax.lax.broadcasted_iota(jnp.int32, (IC, L), 1) + i * L
            for k in range(K):
                vk = jnp.logical_and(mcol >= k, mcol <= (N - 1) + k)
                acc = acc + jnp.where(vk, b_ref[k][:, None], 0.0)
        else:
            acc = acc + jnp.sum(b_ref[...], axis=0)[:, None]
        out_ref[...] = acc

    @pl.when(jnp.logical_and(i > 0, i < num - 1))
    def _fast():
        compute(False)

    @pl.when(jnp.logical_or(i == 0, i == num - 1))
    def _edge():
        compute(True)


def kernel(weights, bias, input_0, input_1, input_2, input_3, input_4,
           input_5, input_6, input_7, input_8, input_9, input_10, input_11,
           input_12, input_13, input_14, input_15, input_16):
    ins = (input_0, input_1, input_2, input_3, input_4, input_5, input_6,
           input_7, input_8, input_9, input_10, input_11, input_12, input_13,
           input_14, input_15, input_16)
    xts = tuple(jnp.transpose(x) for x in ins)

    cur_spec = pl.BlockSpec((IC, L), lambda i: (0, i))
    prev_spec = pl.BlockSpec(
        (IC, HALO), lambda i: (0, jnp.maximum(i * (L // HALO) - 1, 0)))
    outt = pl.pallas_call(
        _body,
        grid=(G,),
        in_specs=[
            pl.BlockSpec((K, OC, IC), lambda i: (0, 0, 0)),
            pl.BlockSpec((K, OC), lambda i: (0, 0)),
        ] + [cur_spec] * K + [prev_spec] * K,
        out_specs=pl.BlockSpec((OC, L), lambda i: (0, i)),
        out_shape=jax.ShapeDtypeStruct((OC, NO), jnp.float32),
        compiler_params=pltpu.CompilerParams(
            dimension_semantics=("arbitrary",)),
    )(weights, bias, *xts, *xts)
    return jnp.transpose(outt)


# speedup vs baseline: 1.2570x; 1.0301x over previous
import jax
import jax.numpy as jnp
from jax.experimental import pallas as pl
from jax.experimental.pallas import tpu as pltpu

K = 17
N = 50000
IC = 32
OC = 32
L = 4096
NO = N + K - 1
G = (NO + L - 1) // L
HALO = 128


def _body(w_ref, b_ref, *refs):
    cur = refs[:K]
    prev = refs[K:2 * K]
    out_ref = refs[2 * K]

    i = pl.program_id(0)
    num = pl.num_programs(0)

    def compute(masked):
        acc = None
        for k in range(K):
            z = jnp.concatenate(
                [prev[k][:, HALO - (K - 1):], cur[k][...]], axis=1)
            sh = jax.lax.slice(z, (0, K - 1 - k), (IC, K - 1 - k + L))
            if masked:
                mcol = jax.lax.broadcasted_iota(jnp.int32, (IC, L), 1) + i * L
                valid = jnp.logical_and(mcol >= k, mcol <= (N - 1) + k)
                sh = jnp.where(valid, sh, 0.0)
            p = jax.lax.dot_general(
                w_ref[k], sh, (((1,), (0,)), ((), ())),
                preferred_element_type=jnp.float32)
            acc = p if acc is None else acc + p
        if masked:
            mcol = jax.lax.broadcasted_iota(jnp.int32, (IC, L), 1) + i * L
            for k in range(K):
                vk = jnp.logical_and(mcol >= k, mcol <= (N - 1) + k)
                acc = acc + jnp.where(vk, b_ref[k][:, None], 0.0)
        else:
            acc = acc + jnp.sum(b_ref[...], axis=0)[:, None]
        out_ref[...] = acc

    @pl.when(jnp.logical_and(i > 0, i < num - 1))
    def _fast():
        compute(False)

    @pl.when(jnp.logical_or(i == 0, i == num - 1))
    def _edge():
        compute(True)


def kernel(weights, bias, input_0, input_1, input_2, input_3, input_4,
           input_5, input_6, input_7, input_8, input_9, input_10, input_11,
           input_12, input_13, input_14, input_15, input_16):
    ins = (input_0, input_1, input_2, input_3, input_4, input_5, input_6,
           input_7, input_8, input_9, input_10, input_11, input_12, input_13,
           input_14, input_15, input_16)
    xts = tuple(jnp.transpose(x) for x in ins)

    cur_spec = pl.BlockSpec((IC, L), lambda i: (0, i))
    prev_spec = pl.BlockSpec(
        (IC, HALO), lambda i: (0, jnp.maximum(i * (L // HALO) - 1, 0)))
    outt = pl.pallas_call(
        _body,
        grid=(G,),
        in_specs=[
            pl.BlockSpec((K, OC, IC), lambda i: (0, 0, 0)),
            pl.BlockSpec((K, OC), lambda i: (0, 0)),
        ] + [cur_spec] * K + [prev_spec] * K,
        out_specs=pl.BlockSpec((OC, L), lambda i: (0, i)),
        out_shape=jax.ShapeDtypeStruct((OC, NO), jnp.float32),
        compiler_params=pltpu.CompilerParams(
            dimension_semantics=("arbitrary",)),
    )(weights, bias, *xts, *xts)
    return jnp.transpose(outt)


# speedup vs baseline: 1.2589x; 1.0016x over previous
import jax
import jax.numpy as jnp
from jax.experimental import pallas as pl
from jax.experimental.pallas import tpu as pltpu

K = 17
N = 50000
IC = 32
OC = 32
L = 4096
NO = N + K - 1
G = (NO + L - 1) // L
HALO = 128


def _body(w_ref, b_ref, *refs):
    cur = refs[:K]
    prev = refs[K:2 * K]
    out_ref = refs[2 * K]

    i = pl.program_id(0)
    num = pl.num_programs(0)

    def compute(masked):
        acc = None
        for k in range(K):
            z = jnp.concatenate(
                [prev[k][:, HALO - (K - 1):], cur[k][...]], axis=1)
            sh = jax.lax.slice(z, (0, K - 1 - k), (IC, K - 1 - k + L))
            if masked:
                mcol = jax.lax.broadcasted_iota(jnp.int32, (IC, L), 1) + i * L
                valid = jnp.logical_and(mcol >= k, mcol <= (N - 1) + k)
                sh = jnp.where(valid, sh, 0.0)
            p = jax.lax.dot_general(
                w_ref[k], sh, (((1,), (0,)), ((), ())),
                preferred_element_type=jnp.float32)
            acc = p if acc is None else acc + p
        if masked:
            mcol = jax.lax.broadcasted_iota(jnp.int32, (IC, L), 1) + i * L
            for k in range(K):
                vk = jnp.logical_and(mcol >= k, mcol <= (N - 1) + k)
                acc = acc + jnp.where(vk, b_ref[k][:, None], 0.0)
        else:
            acc = acc + jnp.sum(b_ref[...], axis=0)[:, None]
        out_ref[...] = acc

    @pl.when(jnp.logical_and(i > 0, i < num - 1))
    def _fast():
        compute(False)

    @pl.when(jnp.logical_or(i == 0, i == num - 1))
    def _edge():
        compute(True)


def kernel(weights, bias, input_0, input_1, input_2, input_3, input_4,
           input_5, input_6, input_7, input_8, input_9, input_10, input_11,
           input_12, input_13, input_14, input_15, input_16):
    ins = (input_0, input_1, input_2, input_3, input_4, input_5, input_6,
           input_7, input_8, input_9, input_10, input_11, input_12, input_13,
           input_14, input_15, input_16)
    xts = tuple(jnp.transpose(x) for x in ins)

    cur_spec = pl.BlockSpec((IC, L), lambda i: (0, i))
    prev_spec = pl.BlockSpec(
        (IC, HALO), lambda i: (0, jnp.maximum(i * (L // HALO) - 1, 0)))
    outt = pl.pallas_call(
        _body,
        grid=(G,),
        in_specs=[
            pl.BlockSpec((K, OC, IC), lambda i: (0, 0, 0)),
            pl.BlockSpec((K, OC), lambda i: (0, 0)),
        ] + [cur_spec] * K + [prev_spec] * K,
        out_specs=pl.BlockSpec((OC, L), lambda i: (0, i)),
        out_shape=jax.ShapeDtypeStruct((OC, NO), jnp.float32),
        compiler_params=pltpu.CompilerParams(
            dimension_semantics=("parallel",)),
    )(weights, bias, *xts, *xts)
    return jnp.transpose(outt)


# speedup vs baseline: 1.5890x; 1.2622x over previous
import jax
import jax.numpy as jnp
from jax.experimental import pallas as pl
from jax.experimental.pallas import tpu as pltpu

K = 17
N = 50000
IC = 32
OC = 32
L = 4096
NO = N + K - 1
G = (NO + L - 1) // L
HALO = 128


def _body(w_ref, b_ref, *refs):
    cur = refs[:K]
    prev = refs[K:2 * K]
    out_ref = refs[2 * K]

    i = pl.program_id(0)
    num = pl.num_programs(0)

    l128 = jax.lax.broadcasted_iota(jnp.int32, (IC, HALO), 1)

    def compute(masked):
        acc_h = None
        acc_t = None
        for k in range(K):
            rc = pltpu.roll(cur[k][...], k, 1)
            rp = pltpu.roll(prev[k][...], k, 1)
            hd = jnp.where(l128 < k, rp, rc[:, :HALO])
            tl = rc[:, HALO:]
            if masked:
                mh = l128 + i * L
                mt = (jax.lax.broadcasted_iota(jnp.int32, (IC, L - HALO), 1)
                      + i * L + HALO)
                hd = jnp.where(
                    jnp.logical_and(mh >= k, mh <= (N - 1) + k), hd, 0.0)
                tl = jnp.where(
                    jnp.logical_and(mt >= k, mt <= (N - 1) + k), tl, 0.0)
            ph = jax.lax.dot_general(
                w_ref[k], hd, (((1,), (0,)), ((), ())),
                preferred_element_type=jnp.float32)
            pt = jax.lax.dot_general(
                w_ref[k], tl, (((1,), (0,)), ((), ())),
                preferred_element_type=jnp.float32)
            acc_h = ph if acc_h is None else acc_h + ph
            acc_t = pt if acc_t is None else acc_t + pt
        if masked:
            mh = l128 + i * L
            mt = (jax.lax.broadcasted_iota(jnp.int32, (IC, L - HALO), 1)
                  + i * L + HALO)
            for k in range(K):
                acc_h = acc_h + jnp.where(
                    jnp.logical_and(mh >= k, mh <= (N - 1) + k),
                    b_ref[k][:, None], 0.0)
                acc_t = acc_t + jnp.where(
                    jnp.logical_and(mt >= k, mt <= (N - 1) + k),
                    b_ref[k][:, None], 0.0)
        else:
            btot = jnp.sum(b_ref[...], axis=0)[:, None]
            acc_h = acc_h + btot
            acc_t = acc_t + btot
        out_ref[:, :HALO] = acc_h
        out_ref[:, HALO:] = acc_t

    @pl.when(jnp.logical_and(i > 0, i < num - 1))
    def _fast():
        compute(False)

    @pl.when(jnp.logical_or(i == 0, i == num - 1))
    def _edge():
        compute(True)


def kernel(weights, bias, input_0, input_1, input_2, input_3, input_4,
           input_5, input_6, input_7, input_8, input_9, input_10, input_11,
           input_12, input_13, input_14, input_15, input_16):
    ins = (input_0, input_1, input_2, input_3, input_4, input_5, input_6,
           input_7, input_8, input_9, input_10, input_11, input_12, input_13,
           input_14, input_15, input_16)
    xts = tuple(jnp.transpose(x) for x in ins)

    cur_spec = pl.BlockSpec((IC, L), lambda i: (0, i))
    prev_spec = pl.BlockSpec(
        (IC, HALO), lambda i: (0, jnp.maximum(i * (L // HALO) - 1, 0)))
    outt = pl.pallas_call(
        _body,
        grid=(G,),
        in_specs=[
            pl.BlockSpec((K, OC, IC), lambda i: (0, 0, 0)),
            pl.BlockSpec((K, OC), lambda i: (0, 0)),
        ] + [cur_spec] * K + [prev_spec] * K,
        out_specs=pl.BlockSpec((OC, L), lambda i: (0, i)),
        out_shape=jax.ShapeDtypeStruct((OC, NO), jnp.float32),
        compiler_params=pltpu.CompilerParams(
            dimension_semantics=("parallel",)),
    )(weights, bias, *xts, *xts)
    return jnp.transpose(outt)
